# baseline (speedup 1.0000x reference)
import functools

import jax
import jax.numpy as jnp
from jax import lax
from jax.experimental import pallas as pl
from jax.experimental.pallas import tpu as pltpu
from jax.experimental.pallas import tpu_sc as plsc

N = 10000
E = 320000
D = 128
HID = 128
DE = 4

CPAD = 16
TW = D + CPAD

NC = 2
NS = 16
NW = NC * NS
NSLICE = 5
ES = E // NSLICE
EPW = ES // NW
CHUNK = 40
NCHUNK = EPW // CHUNK

NBLK = 2000
EBLK = 2000


def _silu(x):
    return x * jax.nn.sigmoid(x)


def _tables_body(h_ref, w1a_ref, w1b_ref, tr_ref, tc_ref):
    h = h_ref[...]
    tr_ref[...] = jnp.dot(h, w1a_ref[...], preferred_element_type=jnp.float32)
    tc_ref[...] = jnp.dot(h, w1b_ref[...], preferred_element_type=jnp.float32)


def _build_tables(h, w1a, w1b):
    return pl.pallas_call(
        _tables_body,
        grid=(N // NBLK,),
        in_specs=[
            pl.BlockSpec((NBLK, D), lambda i: (i, 0)),
            pl.BlockSpec((D, D), lambda i: (0, 0)),
            pl.BlockSpec((D, D), lambda i: (0, 0)),
        ],
        out_specs=[
            pl.BlockSpec((NBLK, D), lambda i: (i, 0)),
            pl.BlockSpec((NBLK, D), lambda i: (i, 0)),
        ],
        out_shape=[
            jax.ShapeDtypeStruct((N, D), jnp.float32),
            jax.ShapeDtypeStruct((N, D), jnp.float32),
        ],
    )(h, w1a, w1b)


_SC_PARAMS = pltpu.CompilerParams(use_tc_tiling_on_sc=False)


def _sc_gather(table_r, table_c, coordp, row, col):
    mesh = plsc.VectorSubcoreMesh(core_axis_name="c", subcore_axis_name="s")
    npair = NCHUNK // 2

    @functools.partial(
        pl.kernel,
        mesh=mesh,
        compiler_params=_SC_PARAMS,
        out_type=[
            jax.ShapeDtypeStruct((ES, D), jnp.float32),
            jax.ShapeDtypeStruct((ES, D), jnp.float32),
            jax.ShapeDtypeStruct((ES, CPAD), jnp.float32),
        ],
        scratch_types=(
            [pltpu.VMEM((CHUNK,), jnp.int32)] * 4
            + [pltpu.VMEM((CHUNK, D), jnp.float32)] * 4
            + [pltpu.VMEM((CHUNK, CPAD), jnp.float32)] * 4
            + [pltpu.SemaphoreType.DMA] * 6
        ),
    )
    def k(tr_hbm, tc_hbm, cp_hbm, row_hbm, col_hbm, gr_hbm, gc_hbm, cd_hbm,
          idx_r0, idx_c0, idx_r1, idx_c1,
          buf_r0, buf_c0, buf_r1, buf_c1,
          cpr0, cpc0, cpr1, cpc1,
          semi0, semi1, semg0, semg1, semw0, semw1):
        wid = lax.axis_index("s") * NC + lax.axis_index("c")
        tile_base = wid * EPW
        sets = (
            (idx_r0, idx_c0, buf_r0, buf_c0, cpr0, cpc0, semi0, semg0, semw0),
            (idx_r1, idx_c1, buf_r1, buf_c1, cpr1, cpc1, semi1, semg1, semw1),
        )

        def start_idx(st, base):
            ir, ic, _, _, _, _, si, _, _ = st
            pltpu.make_async_copy(row_hbm.at[pl.ds(base, CHUNK)], ir, si).start()
            pltpu.make_async_copy(col_hbm.at[pl.ds(base, CHUNK)], ic, si).start()

        def wait_idx(st):
            ir, ic, _, _, _, _, si, _, _ = st
            pltpu.make_async_copy(row_hbm.at[pl.ds(0, CHUNK)], ir, si).wait()
            pltpu.make_async_copy(col_hbm.at[pl.ds(0, CHUNK)], ic, si).wait()

        def start_gathers(st):
            ir, ic, br, bc, pr, pc, _, sg, _ = st
            pltpu.make_async_copy(tr_hbm.at[ir], br, sg).start()
            pltpu.make_async_copy(tc_hbm.at[ic], bc, sg).start()
            pltpu.make_async_copy(cp_hbm.at[ir], pr, sg).start()
            pltpu.make_async_copy(cp_hbm.at[ic], pc, sg).start()

        def wait_gathers(st):
            ir, ic, br, bc, pr, pc, _, sg, _ = st
            pltpu.make_async_copy(tr_hbm.at[ir], br, sg).wait()
            pltpu.make_async_copy(tc_hbm.at[ic], bc, sg).wait()
            pltpu.make_async_copy(cp_hbm.at[ir], pr, sg).wait()
            pltpu.make_async_copy(cp_hbm.at[ic], pc, sg).wait()

        def diff(st):
            _, _, _, _, pr, pc, _, _, _ = st

            @pl.loop(0, CHUNK)
            def _(j):
                pr[j] = pr[j] - pc[j]

        def start_wb(st, base):
            _, _, br, bc, pr, _, _, _, sw = st
            pltpu.make_async_copy(br, gr_hbm.at[pl.ds(base, CHUNK)], sw).start()
            pltpu.make_async_copy(bc, gc_hbm.at[pl.ds(base, CHUNK)], sw).start()
            pltpu.make_async_copy(pr, cd_hbm.at[pl.ds(base, CHUNK)], sw).start()

        def wait_wb(st):
            _, _, br, bc, pr, _, _, _, sw = st
            pltpu.make_async_copy(br, gr_hbm.at[pl.ds(0, CHUNK)], sw).wait()
            pltpu.make_async_copy(bc, gc_hbm.at[pl.ds(0, CHUNK)], sw).wait()
            pltpu.make_async_copy(pr, cd_hbm.at[pl.ds(0, CHUNK)], sw).wait()

        start_idx(sets[0], tile_base)
        start_idx(sets[1], tile_base + CHUNK)

        @pl.loop(0, npair)
        def _(ii):
            base0 = tile_base + (2 * ii) * CHUNK
            base1 = base0 + CHUNK

            @pl.when(ii > 0)
            def _():
                wait_wb(sets[0])
            wait_idx(sets[0])
            start_gathers(sets[0])

            @pl.when(ii > 0)
            def _():
                wait_wb(sets[1])
            wait_idx(sets[1])
            start_gathers(sets[1])

            wait_gathers(sets[0])
            diff(sets[0])
            start_wb(sets[0], base0)

            wait_gathers(sets[1])
            diff(sets[1])
            start_wb(sets[1], base1)

            @pl.when(ii < npair - 1)
            def _():
                start_idx(sets[0], base0 + 2 * CHUNK)
                start_idx(sets[1], base1 + 2 * CHUNK)

        wait_wb(sets[0])
        wait_wb(sets[1])

    return k(table_r, table_c, coordp, row, col)


def _edge_body(gr_ref, gc_ref, cd_ref, ea_ref, w1e_ref, w1r_ref, be1_ref,
               we2_ref, be2_ref, wc1_ref, bc1_ref, wc2_ref,
               ef_ref, trans_ref):
    cd = cd_ref[...]
    radial = jnp.sum(cd * cd, axis=1, keepdims=True)
    norm = jnp.sqrt(radial + 1e-08)
    cdn = cd / (norm + 1.0)
    pre = (gr_ref[...] + gc_ref[...]
           + radial * w1r_ref[...]
           + jnp.dot(ea_ref[...], w1e_ref[...], preferred_element_type=jnp.float32)
           + be1_ref[...])
    m = _silu(pre)
    ef = _silu(jnp.dot(m, we2_ref[...], preferred_element_type=jnp.float32)
               + be2_ref[...])
    c = _silu(jnp.dot(ef, wc1_ref[...], preferred_element_type=jnp.float32)
              + bc1_ref[...])
    coef = jnp.dot(c, wc2_ref[...], preferred_element_type=jnp.float32)
    ef_ref[...] = ef
    trans_ref[...] = cdn * coef


def _edge_mlp(g_r, g_c, cd, edge_attr, w1e, w1r, b_e1, W_e2, b_e2, W_c1, b_c1, W_c2):
    full = lambda shape: pl.BlockSpec(shape, lambda i: tuple(0 for _ in shape))
    return pl.pallas_call(
        _edge_body,
        grid=(ES // EBLK,),
        in_specs=[
            pl.BlockSpec((EBLK, D), lambda i: (i, 0)),
            pl.BlockSpec((EBLK, D), lambda i: (i, 0)),
            pl.BlockSpec((EBLK, CPAD), lambda i: (i, 0)),
            pl.BlockSpec((EBLK, DE), lambda i: (i, 0)),
            full((DE, D)),
            full((1, D)),
            full((1, D)),
            full((D, D)),
            full((1, D)),
            full((D, D)),
            full((1, D)),
            full((D, 1)),
        ],
        out_specs=[
            pl.BlockSpec((EBLK, D), lambda i: (i, 0)),
            pl.BlockSpec((EBLK, CPAD), lambda i: (i, 0)),
        ],
        out_shape=[
            jax.ShapeDtypeStruct((ES, D), jnp.float32),
            jax.ShapeDtypeStruct((ES, CPAD), jnp.float32),
        ],
    )(g_r, g_c, cd, edge_attr, w1e, w1r, b_e1, W_e2, b_e2, W_c1, b_c1, W_c2)


NPW = N // NS


def _sc_scatter(rows, efs, transs, seed_h, seed_c):
    nsl = len(rows)
    mesh = plsc.VectorSubcoreMesh(core_axis_name="c", subcore_axis_name="s")

    @functools.partial(
        pl.kernel,
        mesh=mesh,
        compiler_params=_SC_PARAMS,
        out_type=[
            jax.ShapeDtypeStruct((NC, N, D), jnp.float32),
            jax.ShapeDtypeStruct((NC, N, CPAD), jnp.float32),
        ],
        scratch_types=[
            pltpu.VMEM((CHUNK,), jnp.int32),
            pltpu.VMEM((CHUNK, D), jnp.float32),
            pltpu.VMEM((CHUNK, CPAD), jnp.float32),
            pltpu.VMEM_SHARED((N, D), jnp.float32),
            pltpu.VMEM_SHARED((N, CPAD), jnp.float32),
            pltpu.SemaphoreType.DMA,
        ],
    )
    def k(*refs):
        rows_hbm = refs[0:nsl]
        efs_hbm = refs[nsl:2 * nsl]
        trs_hbm = refs[2 * nsl:3 * nsl]
        sh_hbm, sc_hbm, oh_hbm, oc_hbm = refs[3 * nsl:3 * nsl + 4]
        idx_v, buf_ef, buf_tr, acc_h, acc_c, sem = refs[3 * nsl + 4:]
        cid = lax.axis_index("c")
        sid = lax.axis_index("s")
        wid = sid * NC + cid
        tile_base = wid * EPW

        pltpu.sync_copy(sh_hbm.at[cid].at[pl.ds(sid * NPW, NPW)],
                        acc_h.at[pl.ds(sid * NPW, NPW)])
        pltpu.sync_copy(sc_hbm.at[cid].at[pl.ds(sid * NPW, NPW)],
                        acc_c.at[pl.ds(sid * NPW, NPW)])
        plsc.subcore_barrier()

        for t in range(nsl):
            row_hbm, ef_hbm, tr_hbm = rows_hbm[t], efs_hbm[t], trs_hbm[t]

            @pl.loop(0, NCHUNK)
            def _(i):
                base = tile_base + i * CHUNK
                pltpu.sync_copy(row_hbm.at[pl.ds(base, CHUNK)], idx_v)
                cp_e = pltpu.async_copy(ef_hbm.at[pl.ds(base, CHUNK)], buf_ef, sem)
                cp_t = pltpu.async_copy(tr_hbm.at[pl.ds(base, CHUNK)], buf_tr, sem)
                cp_e.wait()
                cp_t.wait()
                pltpu.sync_copy(buf_ef, acc_h.at[idx_v], add=True)
                pltpu.sync_copy(buf_tr, acc_c.at[idx_v], add=True)

        plsc.subcore_barrier()
        pltpu.sync_copy(acc_h.at[pl.ds(sid * NPW, NPW)],
                        oh_hbm.at[cid].at[pl.ds(sid * NPW, NPW)])
        pltpu.sync_copy(acc_c.at[pl.ds(sid * NPW, NPW)],
                        oc_hbm.at[cid].at[pl.ds(sid * NPW, NPW)])

    return k(*rows, *efs, *transs, seed_h, seed_c)


def _node_body(h_ref, ph0_ref, ph1_ref, pc0_ref, pc1_ref, coord_ref,
               wn1a_ref, wn1b_ref, bn1_ref, wn2_ref, bn2_ref,
               hout_ref, cout_ref):
    h = h_ref[...]
    agg_h = ph0_ref[...] + ph1_ref[...]
    agg_c = pc0_ref[...] + pc1_ref[...]
    pre = (jnp.dot(h, wn1a_ref[...], preferred_element_type=jnp.float32)
           + jnp.dot(agg_h, wn1b_ref[...], preferred_element_type=jnp.float32)
           + bn1_ref[...])
    nm = _silu(pre)
    hout_ref[...] = h + jnp.dot(nm, wn2_ref[...],
                                preferred_element_type=jnp.float32) + bn2_ref[...]
    cout_ref[...] = coord_ref[...] + agg_c[:, :3]


def _node_mlp(h, ph0, ph1, pc0, pc1, coord, wn1a, wn1b, b_n1, W_n2, b_n2):
    full = lambda shape: pl.BlockSpec(shape, lambda i: tuple(0 for _ in shape))
    return pl.pallas_call(
        _node_body,
        grid=(N // NBLK,),
        in_specs=[
            pl.BlockSpec((NBLK, D), lambda i: (i, 0)),
            pl.BlockSpec((NBLK, D), lambda i: (i, 0)),
            pl.BlockSpec((NBLK, D), lambda i: (i, 0)),
            pl.BlockSpec((NBLK, CPAD), lambda i: (i, 0)),
            pl.BlockSpec((NBLK, CPAD), lambda i: (i, 0)),
            pl.BlockSpec((NBLK, 3), lambda i: (i, 0)),
            full((D, D)),
            full((D, D)),
            full((1, D)),
            full((D, D)),
            full((1, D)),
        ],
        out_specs=[
            pl.BlockSpec((NBLK, D), lambda i: (i, 0)),
            pl.BlockSpec((NBLK, 3), lambda i: (i, 0)),
        ],
        out_shape=[
            jax.ShapeDtypeStruct((N, D), jnp.float32),
            jax.ShapeDtypeStruct((N, 3), jnp.float32),
        ],
    )(h, ph0, ph1, pc0, pc1, coord, wn1a, wn1b, b_n1, W_n2, b_n2)


def kernel(h, edge_index, coord, edge_attr,
           W_e1, b_e1, W_e2, b_e2, W_n1, b_n1, W_n2, b_n2, W_c1, b_c1, W_c2):
    row = edge_index[0]
    col = edge_index[1]
    coordp = jnp.pad(coord, ((0, 0), (0, CPAD - 3)))

    w1a = W_e1[:D]
    w1b = W_e1[D:2 * D]
    w1r = W_e1[2 * D:2 * D + 1]
    w1e = W_e1[2 * D + 1:]
    wn1a = W_n1[:D]
    wn1b = W_n1[D:]

    table_r, table_c = _build_tables(h, w1a, w1b)

    zero_h = jnp.zeros((NC, N, D), jnp.float32)
    zero_c = jnp.zeros((NC, N, CPAD), jnp.float32)
    rows_l, efs_l, trs_l = [], [], []
    for s in range(NSLICE):
        row_s = lax.slice_in_dim(row, s * ES, (s + 1) * ES)
        col_s = lax.slice_in_dim(col, s * ES, (s + 1) * ES)
        ea_s = lax.slice_in_dim(edge_attr, s * ES, (s + 1) * ES)
        g_r, g_c, cd = _sc_gather(table_r, table_c, coordp, row_s, col_s)
        ef_s, trans_s = _edge_mlp(g_r, g_c, cd, ea_s, w1e, w1r,
                                  b_e1.reshape(1, -1), W_e2, b_e2.reshape(1, -1),
                                  W_c1, b_c1.reshape(1, -1), W_c2)
        rows_l.append(row_s)
        efs_l.append(ef_s)
        trs_l.append(trans_s)

    part_h, part_c = _sc_scatter(rows_l[:3], efs_l[:3], trs_l[:3],
                                 zero_h, zero_c)
    part_h, part_c = _sc_scatter(rows_l[3:], efs_l[3:], trs_l[3:],
                                 part_h, part_c)
    ef = jnp.concatenate(efs_l, axis=0)
    h_out, coord_out = _node_mlp(h, part_h[0], part_h[1], part_c[0], part_c[1],
                                 coord, wn1a, wn1b, b_n1.reshape(1, -1),
                                 W_n2, b_n2.reshape(1, -1))
    return (h_out, coord_out, ef)


# speedup vs baseline: 1.0987x; 1.0987x over previous
import functools

import jax
import jax.numpy as jnp
from jax import lax
from jax.experimental import pallas as pl
from jax.experimental.pallas import tpu as pltpu
from jax.experimental.pallas import tpu_sc as plsc

N = 10000
E = 320000
D = 128
HID = 128
DE = 4

CPAD = 16
TW = D + CPAD

NC = 2
NS = 16
NW = NC * NS
NSLICE = 5
ES = E // NSLICE
EPW = ES // NW
CHUNK = 40
NCHUNK = EPW // CHUNK
SCHUNK = 80
SNCHUNK = EPW // SCHUNK

NBLK = 2000
EBLK = 2000


def _silu(x):
    return x * jax.nn.sigmoid(x)


def _tables_body(h_ref, w1a_ref, w1b_ref, tr_ref, tc_ref):
    h = h_ref[...]
    tr_ref[...] = jnp.dot(h, w1a_ref[...], preferred_element_type=jnp.float32)
    tc_ref[...] = jnp.dot(h, w1b_ref[...], preferred_element_type=jnp.float32)


def _build_tables(h, w1a, w1b):
    return pl.pallas_call(
        _tables_body,
        grid=(N // NBLK,),
        in_specs=[
            pl.BlockSpec((NBLK, D), lambda i: (i, 0)),
            pl.BlockSpec((D, D), lambda i: (0, 0)),
            pl.BlockSpec((D, D), lambda i: (0, 0)),
        ],
        out_specs=[
            pl.BlockSpec((NBLK, D), lambda i: (i, 0)),
            pl.BlockSpec((NBLK, D), lambda i: (i, 0)),
        ],
        out_shape=[
            jax.ShapeDtypeStruct((N, D), jnp.float32),
            jax.ShapeDtypeStruct((N, D), jnp.float32),
        ],
    )(h, w1a, w1b)


_SC_PARAMS = pltpu.CompilerParams(use_tc_tiling_on_sc=False)


def _sc_gather(table_r, table_c, coordp, row, col):
    mesh = plsc.VectorSubcoreMesh(core_axis_name="c", subcore_axis_name="s")
    npair = NCHUNK // 2

    @functools.partial(
        pl.kernel,
        mesh=mesh,
        compiler_params=_SC_PARAMS,
        out_type=[
            jax.ShapeDtypeStruct((ES, D), jnp.float32),
            jax.ShapeDtypeStruct((ES, D), jnp.float32),
            jax.ShapeDtypeStruct((ES, CPAD), jnp.float32),
        ],
        scratch_types=(
            [pltpu.VMEM((CHUNK,), jnp.int32)] * 4
            + [pltpu.VMEM((CHUNK, D), jnp.float32)] * 4
            + [pltpu.VMEM((CHUNK, CPAD), jnp.float32)] * 4
            + [pltpu.SemaphoreType.DMA] * 6
        ),
    )
    def k(tr_hbm, tc_hbm, cp_hbm, row_hbm, col_hbm, gr_hbm, gc_hbm, cd_hbm,
          idx_r0, idx_c0, idx_r1, idx_c1,
          buf_r0, buf_c0, buf_r1, buf_c1,
          cpr0, cpc0, cpr1, cpc1,
          semi0, semi1, semg0, semg1, semw0, semw1):
        wid = lax.axis_index("s") * NC + lax.axis_index("c")
        tile_base = wid * EPW
        sets = (
            (idx_r0, idx_c0, buf_r0, buf_c0, cpr0, cpc0, semi0, semg0, semw0),
            (idx_r1, idx_c1, buf_r1, buf_c1, cpr1, cpc1, semi1, semg1, semw1),
        )

        def start_idx(st, base):
            ir, ic, _, _, _, _, si, _, _ = st
            pltpu.make_async_copy(row_hbm.at[pl.ds(base, CHUNK)], ir, si).start()
            pltpu.make_async_copy(col_hbm.at[pl.ds(base, CHUNK)], ic, si).start()

        def wait_idx(st):
            ir, ic, _, _, _, _, si, _, _ = st
            pltpu.make_async_copy(row_hbm.at[pl.ds(0, CHUNK)], ir, si).wait()
            pltpu.make_async_copy(col_hbm.at[pl.ds(0, CHUNK)], ic, si).wait()

        def start_gathers(st):
            ir, ic, br, bc, pr, pc, _, sg, _ = st
            pltpu.make_async_copy(tr_hbm.at[ir], br, sg).start()
            pltpu.make_async_copy(tc_hbm.at[ic], bc, sg).start()
            pltpu.make_async_copy(cp_hbm.at[ir], pr, sg).start()
            pltpu.make_async_copy(cp_hbm.at[ic], pc, sg).start()

        def wait_gathers(st):
            ir, ic, br, bc, pr, pc, _, sg, _ = st
            pltpu.make_async_copy(tr_hbm.at[ir], br, sg).wait()
            pltpu.make_async_copy(tc_hbm.at[ic], bc, sg).wait()
            pltpu.make_async_copy(cp_hbm.at[ir], pr, sg).wait()
            pltpu.make_async_copy(cp_hbm.at[ic], pc, sg).wait()

        def diff(st):
            _, _, _, _, pr, pc, _, _, _ = st

            @pl.loop(0, CHUNK)
            def _(j):
                pr[j] = pr[j] - pc[j]

        def start_wb(st, base):
            _, _, br, bc, pr, _, _, _, sw = st
            pltpu.make_async_copy(br, gr_hbm.at[pl.ds(base, CHUNK)], sw).start()
            pltpu.make_async_copy(bc, gc_hbm.at[pl.ds(base, CHUNK)], sw).start()
            pltpu.make_async_copy(pr, cd_hbm.at[pl.ds(base, CHUNK)], sw).start()

        def wait_wb(st):
            _, _, br, bc, pr, _, _, _, sw = st
            pltpu.make_async_copy(br, gr_hbm.at[pl.ds(0, CHUNK)], sw).wait()
            pltpu.make_async_copy(bc, gc_hbm.at[pl.ds(0, CHUNK)], sw).wait()
            pltpu.make_async_copy(pr, cd_hbm.at[pl.ds(0, CHUNK)], sw).wait()

        start_idx(sets[0], tile_base)
        start_idx(sets[1], tile_base + CHUNK)

        @pl.loop(0, npair)
        def _(ii):
            base0 = tile_base + (2 * ii) * CHUNK
            base1 = base0 + CHUNK

            @pl.when(ii > 0)
            def _():
                wait_wb(sets[0])
            wait_idx(sets[0])
            start_gathers(sets[0])

            @pl.when(ii > 0)
            def _():
                wait_wb(sets[1])
            wait_idx(sets[1])
            start_gathers(sets[1])

            wait_gathers(sets[0])
            diff(sets[0])
            start_wb(sets[0], base0)

            wait_gathers(sets[1])
            diff(sets[1])
            start_wb(sets[1], base1)

            @pl.when(ii < npair - 1)
            def _():
                start_idx(sets[0], base0 + 2 * CHUNK)
                start_idx(sets[1], base1 + 2 * CHUNK)

        wait_wb(sets[0])
        wait_wb(sets[1])

    return k(table_r, table_c, coordp, row, col)


def _edge_body(gr_ref, gc_ref, cd_ref, ea_ref, w1e_ref, w1r_ref, be1_ref,
               we2_ref, be2_ref, wc1_ref, bc1_ref, wc2_ref,
               ef_ref, trans_ref):
    cd = cd_ref[...]
    radial = jnp.sum(cd * cd, axis=1, keepdims=True)
    norm = jnp.sqrt(radial + 1e-08)
    cdn = cd / (norm + 1.0)
    pre = (gr_ref[...] + gc_ref[...]
           + radial * w1r_ref[...]
           + jnp.dot(ea_ref[...], w1e_ref[...], preferred_element_type=jnp.float32)
           + be1_ref[...])
    m = _silu(pre)
    ef = _silu(jnp.dot(m, we2_ref[...], preferred_element_type=jnp.float32)
               + be2_ref[...])
    c = _silu(jnp.dot(ef, wc1_ref[...], preferred_element_type=jnp.float32)
              + bc1_ref[...])
    coef = jnp.dot(c, wc2_ref[...], preferred_element_type=jnp.float32)
    ef_ref[...] = ef
    trans_ref[...] = cdn * coef


def _edge_mlp(g_r, g_c, cd, edge_attr, w1e, w1r, b_e1, W_e2, b_e2, W_c1, b_c1, W_c2):
    full = lambda shape: pl.BlockSpec(shape, lambda i: tuple(0 for _ in shape))
    return pl.pallas_call(
        _edge_body,
        grid=(ES // EBLK,),
        in_specs=[
            pl.BlockSpec((EBLK, D), lambda i: (i, 0)),
            pl.BlockSpec((EBLK, D), lambda i: (i, 0)),
            pl.BlockSpec((EBLK, CPAD), lambda i: (i, 0)),
            pl.BlockSpec((EBLK, DE), lambda i: (i, 0)),
            full((DE, D)),
            full((1, D)),
            full((1, D)),
            full((D, D)),
            full((1, D)),
            full((D, D)),
            full((1, D)),
            full((D, 1)),
        ],
        out_specs=[
            pl.BlockSpec((EBLK, D), lambda i: (i, 0)),
            pl.BlockSpec((EBLK, CPAD), lambda i: (i, 0)),
        ],
        out_shape=[
            jax.ShapeDtypeStruct((ES, D), jnp.float32),
            jax.ShapeDtypeStruct((ES, CPAD), jnp.float32),
        ],
    )(g_r, g_c, cd, edge_attr, w1e, w1r, b_e1, W_e2, b_e2, W_c1, b_c1, W_c2)


NPW = N // NS


def _sc_scatter(rows, efs, transs, seed_h, seed_c):
    nsl = len(rows)
    mesh = plsc.VectorSubcoreMesh(core_axis_name="c", subcore_axis_name="s")

    @functools.partial(
        pl.kernel,
        mesh=mesh,
        compiler_params=_SC_PARAMS,
        out_type=[
            jax.ShapeDtypeStruct((NC, N, D), jnp.float32),
            jax.ShapeDtypeStruct((NC, N, CPAD), jnp.float32),
        ],
        scratch_types=[
            pltpu.VMEM((SCHUNK,), jnp.int32),
            pltpu.VMEM((SCHUNK, D), jnp.float32),
            pltpu.VMEM((SCHUNK, CPAD), jnp.float32),
            pltpu.VMEM_SHARED((N, D), jnp.float32),
            pltpu.VMEM_SHARED((N, CPAD), jnp.float32),
            pltpu.SemaphoreType.DMA,
        ],
    )
    def k(*refs):
        rows_hbm = refs[0:nsl]
        efs_hbm = refs[nsl:2 * nsl]
        trs_hbm = refs[2 * nsl:3 * nsl]
        sh_hbm, sc_hbm, oh_hbm, oc_hbm = refs[3 * nsl:3 * nsl + 4]
        idx_v, buf_ef, buf_tr, acc_h, acc_c, sem = refs[3 * nsl + 4:]
        cid = lax.axis_index("c")
        sid = lax.axis_index("s")
        wid = sid * NC + cid
        tile_base = wid * EPW

        pltpu.sync_copy(sh_hbm.at[cid].at[pl.ds(sid * NPW, NPW)],
                        acc_h.at[pl.ds(sid * NPW, NPW)])
        pltpu.sync_copy(sc_hbm.at[cid].at[pl.ds(sid * NPW, NPW)],
                        acc_c.at[pl.ds(sid * NPW, NPW)])
        plsc.subcore_barrier()

        for t in range(nsl):
            row_hbm, ef_hbm, tr_hbm = rows_hbm[t], efs_hbm[t], trs_hbm[t]

            @pl.loop(0, SNCHUNK)
            def _(i):
                base = tile_base + i * SCHUNK
                pltpu.sync_copy(row_hbm.at[pl.ds(base, SCHUNK)], idx_v)
                cp_e = pltpu.async_copy(ef_hbm.at[pl.ds(base, SCHUNK)], buf_ef, sem)
                cp_t = pltpu.async_copy(tr_hbm.at[pl.ds(base, SCHUNK)], buf_tr, sem)
                cp_e.wait()
                cp_t.wait()
                pltpu.sync_copy(buf_ef, acc_h.at[idx_v], add=True)
                pltpu.sync_copy(buf_tr, acc_c.at[idx_v], add=True)

        plsc.subcore_barrier()
        pltpu.sync_copy(acc_h.at[pl.ds(sid * NPW, NPW)],
                        oh_hbm.at[cid].at[pl.ds(sid * NPW, NPW)])
        pltpu.sync_copy(acc_c.at[pl.ds(sid * NPW, NPW)],
                        oc_hbm.at[cid].at[pl.ds(sid * NPW, NPW)])

    return k(*rows, *efs, *transs, seed_h, seed_c)


def _node_body(h_ref, ph0_ref, ph1_ref, pc0_ref, pc1_ref, coord_ref,
               wn1a_ref, wn1b_ref, bn1_ref, wn2_ref, bn2_ref,
               hout_ref, cout_ref):
    h = h_ref[...]
    agg_h = ph0_ref[...] + ph1_ref[...]
    agg_c = pc0_ref[...] + pc1_ref[...]
    pre = (jnp.dot(h, wn1a_ref[...], preferred_element_type=jnp.float32)
           + jnp.dot(agg_h, wn1b_ref[...], preferred_element_type=jnp.float32)
           + bn1_ref[...])
    nm = _silu(pre)
    hout_ref[...] = h + jnp.dot(nm, wn2_ref[...],
                                preferred_element_type=jnp.float32) + bn2_ref[...]
    cout_ref[...] = coord_ref[...] + agg_c[:, :3]


def _node_mlp(h, ph0, ph1, pc0, pc1, coord, wn1a, wn1b, b_n1, W_n2, b_n2):
    full = lambda shape: pl.BlockSpec(shape, lambda i: tuple(0 for _ in shape))
    return pl.pallas_call(
        _node_body,
        grid=(N // NBLK,),
        in_specs=[
            pl.BlockSpec((NBLK, D), lambda i: (i, 0)),
            pl.BlockSpec((NBLK, D), lambda i: (i, 0)),
            pl.BlockSpec((NBLK, D), lambda i: (i, 0)),
            pl.BlockSpec((NBLK, CPAD), lambda i: (i, 0)),
            pl.BlockSpec((NBLK, CPAD), lambda i: (i, 0)),
            pl.BlockSpec((NBLK, 3), lambda i: (i, 0)),
            full((D, D)),
            full((D, D)),
            full((1, D)),
            full((D, D)),
            full((1, D)),
        ],
        out_specs=[
            pl.BlockSpec((NBLK, D), lambda i: (i, 0)),
            pl.BlockSpec((NBLK, 3), lambda i: (i, 0)),
        ],
        out_shape=[
            jax.ShapeDtypeStruct((N, D), jnp.float32),
            jax.ShapeDtypeStruct((N, 3), jnp.float32),
        ],
    )(h, ph0, ph1, pc0, pc1, coord, wn1a, wn1b, b_n1, W_n2, b_n2)


def kernel(h, edge_index, coord, edge_attr,
           W_e1, b_e1, W_e2, b_e2, W_n1, b_n1, W_n2, b_n2, W_c1, b_c1, W_c2):
    row = edge_index[0]
    col = edge_index[1]
    coordp = jnp.pad(coord, ((0, 0), (0, CPAD - 3)))

    w1a = W_e1[:D]
    w1b = W_e1[D:2 * D]
    w1r = W_e1[2 * D:2 * D + 1]
    w1e = W_e1[2 * D + 1:]
    wn1a = W_n1[:D]
    wn1b = W_n1[D:]

    table_r, table_c = _build_tables(h, w1a, w1b)

    zero_h = jnp.zeros((NC, N, D), jnp.float32)
    zero_c = jnp.zeros((NC, N, CPAD), jnp.float32)
    rows_l, efs_l, trs_l = [], [], []
    for s in range(NSLICE):
        row_s = lax.slice_in_dim(row, s * ES, (s + 1) * ES)
        col_s = lax.slice_in_dim(col, s * ES, (s + 1) * ES)
        ea_s = lax.slice_in_dim(edge_attr, s * ES, (s + 1) * ES)
        g_r, g_c, cd = _sc_gather(table_r, table_c, coordp, row_s, col_s)
        ef_s, trans_s = _edge_mlp(g_r, g_c, cd, ea_s, w1e, w1r,
                                  b_e1.reshape(1, -1), W_e2, b_e2.reshape(1, -1),
                                  W_c1, b_c1.reshape(1, -1), W_c2)
        rows_l.append(row_s)
        efs_l.append(ef_s)
        trs_l.append(trans_s)

    part_h, part_c = _sc_scatter(rows_l[:3], efs_l[:3], trs_l[:3],
                                 zero_h, zero_c)
    part_h, part_c = _sc_scatter(rows_l[3:], efs_l[3:], trs_l[3:],
                                 part_h, part_c)
    ef = jnp.concatenate(efs_l, axis=0)
    h_out, coord_out = _node_mlp(h, part_h[0], part_h[1], part_c[0], part_c[1],
                                 coord, wn1a, wn1b, b_n1.reshape(1, -1),
                                 W_n2, b_n2.reshape(1, -1))
    return (h_out, coord_out, ef)


# speedup vs baseline: 1.1274x; 1.0261x over previous
import functools

import jax
import jax.numpy as jnp
from jax import lax
from jax.experimental import pallas as pl
from jax.experimental.pallas import tpu as pltpu
from jax.experimental.pallas import tpu_sc as plsc

N = 10000
E = 320000
D = 128
HID = 128
DE = 4

CPAD = 16
TW = D + CPAD

NC = 2
NS = 16
NW = NC * NS
NSLICE = 5
ES = E // NSLICE
EPW = ES // NW
CHUNK = 40
NCHUNK = EPW // CHUNK
SCHUNK = 40
SNCHUNK = EPW // SCHUNK

NBLK = 2000
EBLK = 2000


def _silu(x):
    return x * jax.nn.sigmoid(x)


def _tables_body(h_ref, w1a_ref, w1b_ref, tr_ref, tc_ref):
    h = h_ref[...]
    tr_ref[...] = jnp.dot(h, w1a_ref[...], preferred_element_type=jnp.float32)
    tc_ref[...] = jnp.dot(h, w1b_ref[...], preferred_element_type=jnp.float32)


def _build_tables(h, w1a, w1b):
    return pl.pallas_call(
        _tables_body,
        grid=(N // NBLK,),
        in_specs=[
            pl.BlockSpec((NBLK, D), lambda i: (i, 0)),
            pl.BlockSpec((D, D), lambda i: (0, 0)),
            pl.BlockSpec((D, D), lambda i: (0, 0)),
        ],
        out_specs=[
            pl.BlockSpec((NBLK, D), lambda i: (i, 0)),
            pl.BlockSpec((NBLK, D), lambda i: (i, 0)),
        ],
        out_shape=[
            jax.ShapeDtypeStruct((N, D), jnp.float32),
            jax.ShapeDtypeStruct((N, D), jnp.float32),
        ],
    )(h, w1a, w1b)


_SC_PARAMS = pltpu.CompilerParams(use_tc_tiling_on_sc=False)


def _sc_gather(table_r, table_c, coordp, row, col):
    mesh = plsc.VectorSubcoreMesh(core_axis_name="c", subcore_axis_name="s")
    npair = NCHUNK // 2

    @functools.partial(
        pl.kernel,
        mesh=mesh,
        compiler_params=_SC_PARAMS,
        out_type=[
            jax.ShapeDtypeStruct((ES, D), jnp.float32),
            jax.ShapeDtypeStruct((ES, D), jnp.float32),
            jax.ShapeDtypeStruct((ES, CPAD), jnp.float32),
        ],
        scratch_types=(
            [pltpu.VMEM((CHUNK,), jnp.int32)] * 4
            + [pltpu.VMEM((CHUNK, D), jnp.float32)] * 4
            + [pltpu.VMEM((CHUNK, CPAD), jnp.float32)] * 4
            + [pltpu.SemaphoreType.DMA] * 6
        ),
    )
    def k(tr_hbm, tc_hbm, cp_hbm, row_hbm, col_hbm, gr_hbm, gc_hbm, cd_hbm,
          idx_r0, idx_c0, idx_r1, idx_c1,
          buf_r0, buf_c0, buf_r1, buf_c1,
          cpr0, cpc0, cpr1, cpc1,
          semi0, semi1, semg0, semg1, semw0, semw1):
        wid = lax.axis_index("s") * NC + lax.axis_index("c")
        tile_base = wid * EPW
        sets = (
            (idx_r0, idx_c0, buf_r0, buf_c0, cpr0, cpc0, semi0, semg0, semw0),
            (idx_r1, idx_c1, buf_r1, buf_c1, cpr1, cpc1, semi1, semg1, semw1),
        )

        def start_idx(st, base):
            ir, ic, _, _, _, _, si, _, _ = st
            pltpu.make_async_copy(row_hbm.at[pl.ds(base, CHUNK)], ir, si).start()
            pltpu.make_async_copy(col_hbm.at[pl.ds(base, CHUNK)], ic, si).start()

        def wait_idx(st):
            ir, ic, _, _, _, _, si, _, _ = st
            pltpu.make_async_copy(row_hbm.at[pl.ds(0, CHUNK)], ir, si).wait()
            pltpu.make_async_copy(col_hbm.at[pl.ds(0, CHUNK)], ic, si).wait()

        def start_gathers(st):
            ir, ic, br, bc, pr, pc, _, sg, _ = st
            pltpu.make_async_copy(tr_hbm.at[ir], br, sg).start()
            pltpu.make_async_copy(tc_hbm.at[ic], bc, sg).start()
            pltpu.make_async_copy(cp_hbm.at[ir], pr, sg).start()
            pltpu.make_async_copy(cp_hbm.at[ic], pc, sg).start()

        def wait_gathers(st):
            ir, ic, br, bc, pr, pc, _, sg, _ = st
            pltpu.make_async_copy(tr_hbm.at[ir], br, sg).wait()
            pltpu.make_async_copy(tc_hbm.at[ic], bc, sg).wait()
            pltpu.make_async_copy(cp_hbm.at[ir], pr, sg).wait()
            pltpu.make_async_copy(cp_hbm.at[ic], pc, sg).wait()

        def diff(st):
            _, _, _, _, pr, pc, _, _, _ = st

            @pl.loop(0, CHUNK)
            def _(j):
                pr[j] = pr[j] - pc[j]

        def start_wb(st, base):
            _, _, br, bc, pr, _, _, _, sw = st
            pltpu.make_async_copy(br, gr_hbm.at[pl.ds(base, CHUNK)], sw).start()
            pltpu.make_async_copy(bc, gc_hbm.at[pl.ds(base, CHUNK)], sw).start()
            pltpu.make_async_copy(pr, cd_hbm.at[pl.ds(base, CHUNK)], sw).start()

        def wait_wb(st):
            _, _, br, bc, pr, _, _, _, sw = st
            pltpu.make_async_copy(br, gr_hbm.at[pl.ds(0, CHUNK)], sw).wait()
            pltpu.make_async_copy(bc, gc_hbm.at[pl.ds(0, CHUNK)], sw).wait()
            pltpu.make_async_copy(pr, cd_hbm.at[pl.ds(0, CHUNK)], sw).wait()

        start_idx(sets[0], tile_base)
        start_idx(sets[1], tile_base + CHUNK)

        @pl.loop(0, npair)
        def _(ii):
            base0 = tile_base + (2 * ii) * CHUNK
            base1 = base0 + CHUNK

            @pl.when(ii > 0)
            def _():
                wait_wb(sets[0])
            wait_idx(sets[0])
            start_gathers(sets[0])

            @pl.when(ii > 0)
            def _():
                wait_wb(sets[1])
            wait_idx(sets[1])
            start_gathers(sets[1])

            wait_gathers(sets[0])
            diff(sets[0])
            start_wb(sets[0], base0)

            wait_gathers(sets[1])
            diff(sets[1])
            start_wb(sets[1], base1)

            @pl.when(ii < npair - 1)
            def _():
                start_idx(sets[0], base0 + 2 * CHUNK)
                start_idx(sets[1], base1 + 2 * CHUNK)

        wait_wb(sets[0])
        wait_wb(sets[1])

    return k(table_r, table_c, coordp, row, col)


def _edge_body(gr_ref, gc_ref, cd_ref, ea_ref, w1e_ref, w1r_ref, be1_ref,
               we2_ref, be2_ref, wc1_ref, bc1_ref, wc2_ref,
               ef_ref, trans_ref):
    cd = cd_ref[...]
    radial = jnp.sum(cd * cd, axis=1, keepdims=True)
    norm = jnp.sqrt(radial + 1e-08)
    cdn = cd / (norm + 1.0)
    pre = (gr_ref[...] + gc_ref[...]
           + radial * w1r_ref[...]
           + jnp.dot(ea_ref[...], w1e_ref[...], preferred_element_type=jnp.float32)
           + be1_ref[...])
    m = _silu(pre)
    ef = _silu(jnp.dot(m, we2_ref[...], preferred_element_type=jnp.float32)
               + be2_ref[...])
    c = _silu(jnp.dot(ef, wc1_ref[...], preferred_element_type=jnp.float32)
              + bc1_ref[...])
    coef = jnp.dot(c, wc2_ref[...], preferred_element_type=jnp.float32)
    ef_ref[...] = ef
    trans_ref[...] = cdn * coef


def _edge_mlp(g_r, g_c, cd, edge_attr, w1e, w1r, b_e1, W_e2, b_e2, W_c1, b_c1, W_c2):
    full = lambda shape: pl.BlockSpec(shape, lambda i: tuple(0 for _ in shape))
    return pl.pallas_call(
        _edge_body,
        grid=(ES // EBLK,),
        in_specs=[
            pl.BlockSpec((EBLK, D), lambda i: (i, 0)),
            pl.BlockSpec((EBLK, D), lambda i: (i, 0)),
            pl.BlockSpec((EBLK, CPAD), lambda i: (i, 0)),
            pl.BlockSpec((EBLK, DE), lambda i: (i, 0)),
            full((DE, D)),
            full((1, D)),
            full((1, D)),
            full((D, D)),
            full((1, D)),
            full((D, D)),
            full((1, D)),
            full((D, 1)),
        ],
        out_specs=[
            pl.BlockSpec((EBLK, D), lambda i: (i, 0)),
            pl.BlockSpec((EBLK, CPAD), lambda i: (i, 0)),
        ],
        out_shape=[
            jax.ShapeDtypeStruct((ES, D), jnp.float32),
            jax.ShapeDtypeStruct((ES, CPAD), jnp.float32),
        ],
    )(g_r, g_c, cd, edge_attr, w1e, w1r, b_e1, W_e2, b_e2, W_c1, b_c1, W_c2)


NPW = N // NS


def _sc_scatter(rows, efs, transs, seed_h, seed_c):
    nsl = len(rows)
    mesh = plsc.VectorSubcoreMesh(core_axis_name="c", subcore_axis_name="s")

    @functools.partial(
        pl.kernel,
        mesh=mesh,
        compiler_params=_SC_PARAMS,
        out_type=[
            jax.ShapeDtypeStruct((NC, N, D), jnp.float32),
            jax.ShapeDtypeStruct((NC, N, CPAD), jnp.float32),
        ],
        scratch_types=(
            [pltpu.VMEM((SCHUNK,), jnp.int32)] * 2
            + [pltpu.VMEM((SCHUNK, D), jnp.float32)] * 2
            + [pltpu.VMEM((SCHUNK, CPAD), jnp.float32)] * 2
            + [pltpu.VMEM_SHARED((N, D), jnp.float32),
               pltpu.VMEM_SHARED((N, CPAD), jnp.float32)]
            + [pltpu.SemaphoreType.DMA] * 4
        ),
    )
    def k(*refs):
        rows_hbm = refs[0:nsl]
        efs_hbm = refs[nsl:2 * nsl]
        trs_hbm = refs[2 * nsl:3 * nsl]
        sh_hbm, sc_hbm, oh_hbm, oc_hbm = refs[3 * nsl:3 * nsl + 4]
        (idx0, idx1, ef0, ef1, tr0, tr1, acc_h, acc_c,
         seml0, seml1, sema0, sema1) = refs[3 * nsl + 4:]
        cid = lax.axis_index("c")
        sid = lax.axis_index("s")
        wid = sid * NC + cid
        tile_base = wid * EPW
        sets = ((idx0, ef0, tr0, seml0, sema0), (idx1, ef1, tr1, seml1, sema1))
        npair = SNCHUNK // 2

        pltpu.sync_copy(sh_hbm.at[cid].at[pl.ds(sid * NPW, NPW)],
                        acc_h.at[pl.ds(sid * NPW, NPW)])
        pltpu.sync_copy(sc_hbm.at[cid].at[pl.ds(sid * NPW, NPW)],
                        acc_c.at[pl.ds(sid * NPW, NPW)])
        plsc.subcore_barrier()

        for t in range(nsl):
            row_hbm, ef_hbm, tr_hbm = rows_hbm[t], efs_hbm[t], trs_hbm[t]

            def start_loads(st, base):
                iv, be, bt, sl, _ = st
                pltpu.make_async_copy(
                    row_hbm.at[pl.ds(base, SCHUNK)], iv, sl).start()
                pltpu.make_async_copy(
                    ef_hbm.at[pl.ds(base, SCHUNK)], be, sl).start()
                pltpu.make_async_copy(
                    tr_hbm.at[pl.ds(base, SCHUNK)], bt, sl).start()

            def wait_loads(st):
                iv, be, bt, sl, _ = st
                pltpu.make_async_copy(
                    row_hbm.at[pl.ds(0, SCHUNK)], iv, sl).wait()
                pltpu.make_async_copy(
                    ef_hbm.at[pl.ds(0, SCHUNK)], be, sl).wait()
                pltpu.make_async_copy(
                    tr_hbm.at[pl.ds(0, SCHUNK)], bt, sl).wait()

            start_loads(sets[0], tile_base)
            start_loads(sets[1], tile_base + SCHUNK)

            @pl.loop(0, npair)
            def _(ii):
                base0 = tile_base + (2 * ii) * SCHUNK
                base1 = base0 + SCHUNK

                wait_loads(sets[0])
                iv0, be0, bt0, _, sa0 = sets[0]
                a0 = pltpu.async_copy(be0, acc_h.at[iv0], sa0, add=True)
                a1 = pltpu.async_copy(bt0, acc_c.at[iv0], sa0, add=True)

                wait_loads(sets[1])
                iv1, be1, bt1, _, sa1 = sets[1]
                b0 = pltpu.async_copy(be1, acc_h.at[iv1], sa1, add=True)
                b1 = pltpu.async_copy(bt1, acc_c.at[iv1], sa1, add=True)

                a0.wait()
                a1.wait()

                @pl.when(ii < npair - 1)
                def _():
                    start_loads(sets[0], base0 + 2 * SCHUNK)

                b0.wait()
                b1.wait()

                @pl.when(ii < npair - 1)
                def _():
                    start_loads(sets[1], base1 + 2 * SCHUNK)

        plsc.subcore_barrier()
        pltpu.sync_copy(acc_h.at[pl.ds(sid * NPW, NPW)],
                        oh_hbm.at[cid].at[pl.ds(sid * NPW, NPW)])
        pltpu.sync_copy(acc_c.at[pl.ds(sid * NPW, NPW)],
                        oc_hbm.at[cid].at[pl.ds(sid * NPW, NPW)])

    return k(*rows, *efs, *transs, seed_h, seed_c)


def _node_body(h_ref, ph0_ref, ph1_ref, pc0_ref, pc1_ref, coord_ref,
               wn1a_ref, wn1b_ref, bn1_ref, wn2_ref, bn2_ref,
               hout_ref, cout_ref):
    h = h_ref[...]
    agg_h = ph0_ref[...] + ph1_ref[...]
    agg_c = pc0_ref[...] + pc1_ref[...]
    pre = (jnp.dot(h, wn1a_ref[...], preferred_element_type=jnp.float32)
           + jnp.dot(agg_h, wn1b_ref[...], preferred_element_type=jnp.float32)
           + bn1_ref[...])
    nm = _silu(pre)
    hout_ref[...] = h + jnp.dot(nm, wn2_ref[...],
                                preferred_element_type=jnp.float32) + bn2_ref[...]
    cout_ref[...] = coord_ref[...] + agg_c[:, :3]


def _node_mlp(h, ph0, ph1, pc0, pc1, coord, wn1a, wn1b, b_n1, W_n2, b_n2):
    full = lambda shape: pl.BlockSpec(shape, lambda i: tuple(0 for _ in shape))
    return pl.pallas_call(
        _node_body,
        grid=(N // NBLK,),
        in_specs=[
            pl.BlockSpec((NBLK, D), lambda i: (i, 0)),
            pl.BlockSpec((NBLK, D), lambda i: (i, 0)),
            pl.BlockSpec((NBLK, D), lambda i: (i, 0)),
            pl.BlockSpec((NBLK, CPAD), lambda i: (i, 0)),
            pl.BlockSpec((NBLK, CPAD), lambda i: (i, 0)),
            pl.BlockSpec((NBLK, 3), lambda i: (i, 0)),
            full((D, D)),
            full((D, D)),
            full((1, D)),
            full((D, D)),
            full((1, D)),
        ],
        out_specs=[
            pl.BlockSpec((NBLK, D), lambda i: (i, 0)),
            pl.BlockSpec((NBLK, 3), lambda i: (i, 0)),
        ],
        out_shape=[
            jax.ShapeDtypeStruct((N, D), jnp.float32),
            jax.ShapeDtypeStruct((N, 3), jnp.float32),
        ],
    )(h, ph0, ph1, pc0, pc1, coord, wn1a, wn1b, b_n1, W_n2, b_n2)


def kernel(h, edge_index, coord, edge_attr,
           W_e1, b_e1, W_e2, b_e2, W_n1, b_n1, W_n2, b_n2, W_c1, b_c1, W_c2):
    row = edge_index[0]
    col = edge_index[1]
    coordp = jnp.pad(coord, ((0, 0), (0, CPAD - 3)))

    w1a = W_e1[:D]
    w1b = W_e1[D:2 * D]
    w1r = W_e1[2 * D:2 * D + 1]
    w1e = W_e1[2 * D + 1:]
    wn1a = W_n1[:D]
    wn1b = W_n1[D:]

    table_r, table_c = _build_tables(h, w1a, w1b)

    zero_h = jnp.zeros((NC, N, D), jnp.float32)
    zero_c = jnp.zeros((NC, N, CPAD), jnp.float32)
    rows_l, efs_l, trs_l = [], [], []
    for s in range(NSLICE):
        row_s = lax.slice_in_dim(row, s * ES, (s + 1) * ES)
        col_s = lax.slice_in_dim(col, s * ES, (s + 1) * ES)
        ea_s = lax.slice_in_dim(edge_attr, s * ES, (s + 1) * ES)
        g_r, g_c, cd = _sc_gather(table_r, table_c, coordp, row_s, col_s)
        ef_s, trans_s = _edge_mlp(g_r, g_c, cd, ea_s, w1e, w1r,
                                  b_e1.reshape(1, -1), W_e2, b_e2.reshape(1, -1),
                                  W_c1, b_c1.reshape(1, -1), W_c2)
        rows_l.append(row_s)
        efs_l.append(ef_s)
        trs_l.append(trans_s)

    part_h, part_c = _sc_scatter(rows_l[:3], efs_l[:3], trs_l[:3],
                                 zero_h, zero_c)
    part_h, part_c = _sc_scatter(rows_l[3:], efs_l[3:], trs_l[3:],
                                 part_h, part_c)
    ef = jnp.concatenate(efs_l, axis=0)
    h_out, coord_out = _node_mlp(h, part_h[0], part_h[1], part_c[0], part_c[1],
                                 coord, wn1a, wn1b, b_n1.reshape(1, -1),
                                 W_n2, b_n2.reshape(1, -1))
    return (h_out, coord_out, ef)


# speedup vs baseline: 1.1379x; 1.0093x over previous
import functools

import jax
import jax.numpy as jnp
from jax import lax
from jax.experimental import pallas as pl
from jax.experimental.pallas import tpu as pltpu
from jax.experimental.pallas import tpu_sc as plsc

N = 10000
E = 320000
D = 128
HID = 128
DE = 4

CPAD = 16
TW = D + CPAD

NC = 2
NS = 16
NW = NC * NS
NSLICE = 5
ES = E // NSLICE
EPW = ES // NW
CHUNK = 40
NCHUNK = EPW // CHUNK
SCHUNK = 40
SNCHUNK = EPW // SCHUNK

NBLK = 2000
EBLK = 2000


def _silu(x):
    return x * jax.nn.sigmoid(x)


def _tables_body(h_ref, w1a_ref, w1b_ref, tr_ref, tc_ref):
    h = h_ref[...]
    tr_ref[...] = jnp.dot(h, w1a_ref[...], preferred_element_type=jnp.float32)
    tc_ref[...] = jnp.dot(h, w1b_ref[...], preferred_element_type=jnp.float32)


def _build_tables(h, w1a, w1b):
    return pl.pallas_call(
        _tables_body,
        grid=(N // NBLK,),
        in_specs=[
            pl.BlockSpec((NBLK, D), lambda i: (i, 0)),
            pl.BlockSpec((D, D), lambda i: (0, 0)),
            pl.BlockSpec((D, D), lambda i: (0, 0)),
        ],
        out_specs=[
            pl.BlockSpec((NBLK, D), lambda i: (i, 0)),
            pl.BlockSpec((NBLK, D), lambda i: (i, 0)),
        ],
        out_shape=[
            jax.ShapeDtypeStruct((N, D), jnp.float32),
            jax.ShapeDtypeStruct((N, D), jnp.float32),
        ],
    )(h, w1a, w1b)


_SC_PARAMS = pltpu.CompilerParams(use_tc_tiling_on_sc=False)


def _sc_gather(table_r, table_c, coordp, row, col):
    mesh = plsc.VectorSubcoreMesh(core_axis_name="c", subcore_axis_name="s")
    npair = NCHUNK // 2

    @functools.partial(
        pl.kernel,
        mesh=mesh,
        compiler_params=_SC_PARAMS,
        out_type=[
            jax.ShapeDtypeStruct((ES, D), jnp.float32),
            jax.ShapeDtypeStruct((ES, D), jnp.float32),
            jax.ShapeDtypeStruct((ES, CPAD), jnp.float32),
        ],
        scratch_types=(
            [pltpu.VMEM((CHUNK,), jnp.int32)] * 4
            + [pltpu.VMEM((CHUNK, D), jnp.float32)] * 4
            + [pltpu.VMEM((CHUNK, CPAD), jnp.float32)] * 4
            + [pltpu.SemaphoreType.DMA] * 6
        ),
    )
    def k(tr_hbm, tc_hbm, cp_hbm, row_hbm, col_hbm, gr_hbm, gc_hbm, cd_hbm,
          idx_r0, idx_c0, idx_r1, idx_c1,
          buf_r0, buf_c0, buf_r1, buf_c1,
          cpr0, cpc0, cpr1, cpc1,
          semi0, semi1, semg0, semg1, semw0, semw1):
        wid = lax.axis_index("s") * NC + lax.axis_index("c")
        tile_base = wid * EPW
        sets = (
            (idx_r0, idx_c0, buf_r0, buf_c0, cpr0, cpc0, semi0, semg0, semw0),
            (idx_r1, idx_c1, buf_r1, buf_c1, cpr1, cpc1, semi1, semg1, semw1),
        )

        def start_idx(st, base):
            ir, ic, _, _, _, _, si, _, _ = st
            pltpu.make_async_copy(row_hbm.at[pl.ds(base, CHUNK)], ir, si).start()
            pltpu.make_async_copy(col_hbm.at[pl.ds(base, CHUNK)], ic, si).start()

        def wait_idx(st):
            ir, ic, _, _, _, _, si, _, _ = st
            pltpu.make_async_copy(row_hbm.at[pl.ds(0, CHUNK)], ir, si).wait()
            pltpu.make_async_copy(col_hbm.at[pl.ds(0, CHUNK)], ic, si).wait()

        def start_gathers(st):
            ir, ic, br, bc, pr, pc, _, sg, _ = st
            pltpu.make_async_copy(tr_hbm.at[ir], br, sg).start()
            pltpu.make_async_copy(tc_hbm.at[ic], bc, sg).start()
            pltpu.make_async_copy(cp_hbm.at[ir], pr, sg).start()
            pltpu.make_async_copy(cp_hbm.at[ic], pc, sg).start()

        def wait_gathers(st):
            ir, ic, br, bc, pr, pc, _, sg, _ = st
            pltpu.make_async_copy(tr_hbm.at[ir], br, sg).wait()
            pltpu.make_async_copy(tc_hbm.at[ic], bc, sg).wait()
            pltpu.make_async_copy(cp_hbm.at[ir], pr, sg).wait()
            pltpu.make_async_copy(cp_hbm.at[ic], pc, sg).wait()

        def diff(st):
            _, _, _, _, pr, pc, _, _, _ = st

            @pl.loop(0, CHUNK)
            def _(j):
                pr[j] = pr[j] - pc[j]

        def start_wb(st, base):
            _, _, br, bc, pr, _, _, _, sw = st
            pltpu.make_async_copy(br, gr_hbm.at[pl.ds(base, CHUNK)], sw).start()
            pltpu.make_async_copy(bc, gc_hbm.at[pl.ds(base, CHUNK)], sw).start()
            pltpu.make_async_copy(pr, cd_hbm.at[pl.ds(base, CHUNK)], sw).start()

        def wait_wb(st):
            _, _, br, bc, pr, _, _, _, sw = st
            pltpu.make_async_copy(br, gr_hbm.at[pl.ds(0, CHUNK)], sw).wait()
            pltpu.make_async_copy(bc, gc_hbm.at[pl.ds(0, CHUNK)], sw).wait()
            pltpu.make_async_copy(pr, cd_hbm.at[pl.ds(0, CHUNK)], sw).wait()

        start_idx(sets[0], tile_base)
        start_idx(sets[1], tile_base + CHUNK)

        @pl.loop(0, npair)
        def _(ii):
            base0 = tile_base + (2 * ii) * CHUNK
            base1 = base0 + CHUNK

            @pl.when(ii > 0)
            def _():
                wait_wb(sets[0])
            wait_idx(sets[0])
            start_gathers(sets[0])

            @pl.when(ii > 0)
            def _():
                wait_wb(sets[1])
            wait_idx(sets[1])
            start_gathers(sets[1])

            wait_gathers(sets[0])
            diff(sets[0])
            start_wb(sets[0], base0)

            wait_gathers(sets[1])
            diff(sets[1])
            start_wb(sets[1], base1)

            @pl.when(ii < npair - 1)
            def _():
                start_idx(sets[0], base0 + 2 * CHUNK)
                start_idx(sets[1], base1 + 2 * CHUNK)

        wait_wb(sets[0])
        wait_wb(sets[1])

    return k(table_r, table_c, coordp, row, col)


def _edge_body(gr_ref, gc_ref, cd_ref, ea_ref, w1e_ref, w1r_ref, be1_ref,
               we2_ref, be2_ref, wc1_ref, bc1_ref, wc2_ref,
               ef_ref, trans_ref):
    cd = cd_ref[...]
    radial = jnp.sum(cd * cd, axis=1, keepdims=True)
    norm = jnp.sqrt(radial + 1e-08)
    cdn = cd / (norm + 1.0)
    pre = (gr_ref[...] + gc_ref[...]
           + radial * w1r_ref[...]
           + jnp.dot(ea_ref[...], w1e_ref[...], preferred_element_type=jnp.float32)
           + be1_ref[...])
    m = _silu(pre)
    ef = _silu(jnp.dot(m, we2_ref[...], preferred_element_type=jnp.float32)
               + be2_ref[...])
    c = _silu(jnp.dot(ef, wc1_ref[...], preferred_element_type=jnp.float32)
              + bc1_ref[...])
    coef = jnp.dot(c, wc2_ref[...], preferred_element_type=jnp.float32)
    ef_ref[...] = ef
    trans_ref[...] = cdn * coef


def _edge_mlp(g_r, g_c, cd, edge_attr, w1e, w1r, b_e1, W_e2, b_e2, W_c1, b_c1, W_c2):
    full = lambda shape: pl.BlockSpec(shape, lambda i: tuple(0 for _ in shape))
    return pl.pallas_call(
        _edge_body,
        grid=(ES // EBLK,),
        in_specs=[
            pl.BlockSpec((EBLK, D), lambda i: (i, 0)),
            pl.BlockSpec((EBLK, D), lambda i: (i, 0)),
            pl.BlockSpec((EBLK, CPAD), lambda i: (i, 0)),
            pl.BlockSpec((EBLK, DE), lambda i: (i, 0)),
            full((DE, D)),
            full((1, D)),
            full((1, D)),
            full((D, D)),
            full((1, D)),
            full((D, D)),
            full((1, D)),
            full((D, 1)),
        ],
        out_specs=[
            pl.BlockSpec((EBLK, D), lambda i: (i, 0)),
            pl.BlockSpec((EBLK, CPAD), lambda i: (i, 0)),
        ],
        out_shape=[
            jax.ShapeDtypeStruct((ES, D), jnp.float32),
            jax.ShapeDtypeStruct((ES, CPAD), jnp.float32),
        ],
    )(g_r, g_c, cd, edge_attr, w1e, w1r, b_e1, W_e2, b_e2, W_c1, b_c1, W_c2)


NPW = N // NS


def _sc_scatter(rows, efs, transs, seed_h, seed_c):
    nsl = len(rows)
    mesh = plsc.VectorSubcoreMesh(core_axis_name="c", subcore_axis_name="s")

    @functools.partial(
        pl.kernel,
        mesh=mesh,
        compiler_params=_SC_PARAMS,
        out_type=[
            jax.ShapeDtypeStruct((NC, N, D), jnp.float32),
            jax.ShapeDtypeStruct((NC, N, CPAD), jnp.float32),
        ],
        scratch_types=(
            [pltpu.VMEM((SCHUNK,), jnp.int32)] * 2
            + [pltpu.VMEM((SCHUNK, D), jnp.float32)] * 2
            + [pltpu.VMEM((SCHUNK, CPAD), jnp.float32)] * 2
            + [pltpu.VMEM_SHARED((N, D), jnp.float32),
               pltpu.VMEM_SHARED((N, CPAD), jnp.float32)]
            + [pltpu.SemaphoreType.DMA] * 4
        ),
    )
    def k(*refs):
        rows_hbm = refs[0:nsl]
        efs_hbm = refs[nsl:2 * nsl]
        trs_hbm = refs[2 * nsl:3 * nsl]
        sh_hbm, sc_hbm, oh_hbm, oc_hbm = refs[3 * nsl:3 * nsl + 4]
        (idx0, idx1, ef0, ef1, tr0, tr1, acc_h, acc_c,
         seml0, seml1, sema0, sema1) = refs[3 * nsl + 4:]
        cid = lax.axis_index("c")
        sid = lax.axis_index("s")
        wid = sid * NC + cid
        tile_base = wid * EPW
        sets = ((idx0, ef0, tr0, seml0, sema0), (idx1, ef1, tr1, seml1, sema1))
        npair = SNCHUNK // 2

        pltpu.sync_copy(sh_hbm.at[cid].at[pl.ds(sid * NPW, NPW)],
                        acc_h.at[pl.ds(sid * NPW, NPW)])
        pltpu.sync_copy(sc_hbm.at[cid].at[pl.ds(sid * NPW, NPW)],
                        acc_c.at[pl.ds(sid * NPW, NPW)])
        plsc.subcore_barrier()

        for t in range(nsl):
            row_hbm, ef_hbm, tr_hbm = rows_hbm[t], efs_hbm[t], trs_hbm[t]

            def start_loads(st, base):
                iv, be, bt, sl, _ = st
                pltpu.make_async_copy(
                    row_hbm.at[pl.ds(base, SCHUNK)], iv, sl).start()
                pltpu.make_async_copy(
                    ef_hbm.at[pl.ds(base, SCHUNK)], be, sl).start()
                pltpu.make_async_copy(
                    tr_hbm.at[pl.ds(base, SCHUNK)], bt, sl).start()

            def wait_loads(st):
                iv, be, bt, sl, _ = st
                pltpu.make_async_copy(
                    row_hbm.at[pl.ds(0, SCHUNK)], iv, sl).wait()
                pltpu.make_async_copy(
                    ef_hbm.at[pl.ds(0, SCHUNK)], be, sl).wait()
                pltpu.make_async_copy(
                    tr_hbm.at[pl.ds(0, SCHUNK)], bt, sl).wait()

            start_loads(sets[0], tile_base)
            start_loads(sets[1], tile_base + SCHUNK)

            @pl.loop(0, npair)
            def _(ii):
                base0 = tile_base + (2 * ii) * SCHUNK
                base1 = base0 + SCHUNK

                wait_loads(sets[0])
                iv0, be0, bt0, _, sa0 = sets[0]
                a0 = pltpu.async_copy(be0, acc_h.at[iv0], sa0, add=True)
                a1 = pltpu.async_copy(bt0, acc_c.at[iv0], sa0, add=True)

                wait_loads(sets[1])
                iv1, be1, bt1, _, sa1 = sets[1]
                b0 = pltpu.async_copy(be1, acc_h.at[iv1], sa1, add=True)
                b1 = pltpu.async_copy(bt1, acc_c.at[iv1], sa1, add=True)

                a0.wait()
                a1.wait()

                @pl.when(ii < npair - 1)
                def _():
                    start_loads(sets[0], base0 + 2 * SCHUNK)

                b0.wait()
                b1.wait()

                @pl.when(ii < npair - 1)
                def _():
                    start_loads(sets[1], base1 + 2 * SCHUNK)

        plsc.subcore_barrier()
        pltpu.sync_copy(acc_h.at[pl.ds(sid * NPW, NPW)],
                        oh_hbm.at[cid].at[pl.ds(sid * NPW, NPW)])
        pltpu.sync_copy(acc_c.at[pl.ds(sid * NPW, NPW)],
                        oc_hbm.at[cid].at[pl.ds(sid * NPW, NPW)])

    return k(*rows, *efs, *transs, seed_h, seed_c)


def _node_body(h_ref, ph0_ref, ph1_ref, pc0_ref, pc1_ref, coord_ref,
               wn1a_ref, wn1b_ref, bn1_ref, wn2_ref, bn2_ref,
               hout_ref, cout_ref):
    h = h_ref[...]
    agg_h = ph0_ref[...] + ph1_ref[...]
    agg_c = pc0_ref[...] + pc1_ref[...]
    pre = (jnp.dot(h, wn1a_ref[...], preferred_element_type=jnp.float32)
           + jnp.dot(agg_h, wn1b_ref[...], preferred_element_type=jnp.float32)
           + bn1_ref[...])
    nm = _silu(pre)
    hout_ref[...] = h + jnp.dot(nm, wn2_ref[...],
                                preferred_element_type=jnp.float32) + bn2_ref[...]
    cout_ref[...] = coord_ref[...] + agg_c[:, :3]


def _node_mlp(h, ph0, ph1, pc0, pc1, coord, wn1a, wn1b, b_n1, W_n2, b_n2):
    full = lambda shape: pl.BlockSpec(shape, lambda i: tuple(0 for _ in shape))
    return pl.pallas_call(
        _node_body,
        grid=(N // NBLK,),
        in_specs=[
            pl.BlockSpec((NBLK, D), lambda i: (i, 0)),
            pl.BlockSpec((NBLK, D), lambda i: (i, 0)),
            pl.BlockSpec((NBLK, D), lambda i: (i, 0)),
            pl.BlockSpec((NBLK, CPAD), lambda i: (i, 0)),
            pl.BlockSpec((NBLK, CPAD), lambda i: (i, 0)),
            pl.BlockSpec((NBLK, 3), lambda i: (i, 0)),
            full((D, D)),
            full((D, D)),
            full((1, D)),
            full((D, D)),
            full((1, D)),
        ],
        out_specs=[
            pl.BlockSpec((NBLK, D), lambda i: (i, 0)),
            pl.BlockSpec((NBLK, 3), lambda i: (i, 0)),
        ],
        out_shape=[
            jax.ShapeDtypeStruct((N, D), jnp.float32),
            jax.ShapeDtypeStruct((N, 3), jnp.float32),
        ],
    )(h, ph0, ph1, pc0, pc1, coord, wn1a, wn1b, b_n1, W_n2, b_n2)


def kernel(h, edge_index, coord, edge_attr,
           W_e1, b_e1, W_e2, b_e2, W_n1, b_n1, W_n2, b_n2, W_c1, b_c1, W_c2):
    row = edge_index[0]
    col = edge_index[1]
    coordp = jnp.pad(coord, ((0, 0), (0, CPAD - 3)))

    w1a = W_e1[:D]
    w1b = W_e1[D:2 * D]
    w1r = W_e1[2 * D:2 * D + 1]
    w1e = W_e1[2 * D + 1:]
    wn1a = W_n1[:D]
    wn1b = W_n1[D:]

    table_r, table_c = _build_tables(h, w1a, w1b)

    zero_h = jnp.zeros((NC, N, D), jnp.float32)
    zero_c = jnp.zeros((NC, N, CPAD), jnp.float32)
    rows_l, efs_l, trs_l = [], [], []
    for s in range(NSLICE):
        row_s = lax.slice_in_dim(row, s * ES, (s + 1) * ES)
        col_s = lax.slice_in_dim(col, s * ES, (s + 1) * ES)
        ea_s = lax.slice_in_dim(edge_attr, s * ES, (s + 1) * ES)
        g_r, g_c, cd = _sc_gather(table_r, table_c, coordp, row_s, col_s)
        ef_s, trans_s = _edge_mlp(g_r, g_c, cd, ea_s, w1e, w1r,
                                  b_e1.reshape(1, -1), W_e2, b_e2.reshape(1, -1),
                                  W_c1, b_c1.reshape(1, -1), W_c2)
        rows_l.append(row_s)
        efs_l.append(ef_s)
        trs_l.append(trans_s)

    part_h, part_c = _sc_scatter(rows_l[:2], efs_l[:2], trs_l[:2],
                                 zero_h, zero_c)
    part_h, part_c = _sc_scatter(rows_l[2:4], efs_l[2:4], trs_l[2:4],
                                 part_h, part_c)
    part_h, part_c = _sc_scatter(rows_l[4:], efs_l[4:], trs_l[4:],
                                 part_h, part_c)
    ef = jnp.concatenate(efs_l, axis=0)
    h_out, coord_out = _node_mlp(h, part_h[0], part_h[1], part_c[0], part_c[1],
                                 coord, wn1a, wn1b, b_n1.reshape(1, -1),
                                 W_n2, b_n2.reshape(1, -1))
    return (h_out, coord_out, ef)
